# baseline (speedup 1.0000x reference)
import functools

import jax
import jax.numpy as jnp
from jax import lax
from jax.experimental import pallas as pl
from jax.experimental.pallas import tpu as pltpu
from jax.experimental.pallas import tpu_sc as plsc

EMBED_DIM = 16
BATCH = 16384

_info = plsc.get_sparse_core_info()
_NC, _NS = _info.num_cores, _info.num_subcores
_NW = _NC * _NS
_BPW = BATCH // _NW
_CHUNK = 128
_NCHUNK = _BPW // _CHUNK

_mesh = plsc.VectorSubcoreMesh(core_axis_name="c", subcore_axis_name="s")


@functools.partial(
    pl.kernel,
    mesh=_mesh,
    out_type=jax.ShapeDtypeStruct((_NW, _NCHUNK, _CHUNK, EMBED_DIM), jnp.float32),
    scratch_types=[
        pltpu.VMEM((_NCHUNK, _CHUNK), jnp.int32),
        pltpu.VMEM((_NCHUNK, _CHUNK, EMBED_DIM), jnp.float32),
        pltpu.SemaphoreType.DMA,
    ],
    compiler_params=pltpu.CompilerParams(use_tc_tiling_on_sc=False),
)
def _gather_kernel(table_hbm, idx_hbm, out_hbm, idx_v, rows_v, sem):
    wid = lax.axis_index("s") * _NC + lax.axis_index("c")
    pltpu.sync_copy(idx_hbm.at[wid], idx_v)
    copies = [
        pltpu.async_copy(table_hbm.at[idx_v.at[j]], rows_v.at[j], sem)
        for j in range(_NCHUNK)
    ]
    for c in copies:
        c.wait()
    pltpu.sync_copy(rows_v, out_hbm.at[wid])


def kernel(g, h, r, norm, node_attri):
    idx = h.reshape(_NW, _NCHUNK, _CHUNK)
    out = _gather_kernel(node_attri, idx)
    return out.reshape(BATCH, EMBED_DIM)


# speedup vs baseline: 5.5309x; 5.5309x over previous
import functools

import jax
import jax.numpy as jnp
from jax import lax
from jax.experimental import pallas as pl
from jax.experimental.pallas import tpu as pltpu
from jax.experimental.pallas import tpu_sc as plsc

EMBED_DIM = 16
BATCH = 16384
LANES = 16

_info = plsc.get_sparse_core_info()
_NC, _NS = _info.num_cores, _info.num_subcores
_NW = _NC * _NS
_BPW = BATCH // _NW
_WAVE = 32
_NWAVE = _BPW // _WAVE

_mesh = plsc.VectorSubcoreMesh(core_axis_name="c", subcore_axis_name="s")


@functools.partial(
    pl.kernel,
    mesh=_mesh,
    out_type=jax.ShapeDtypeStruct((EMBED_DIM, BATCH), jnp.float32),
    scratch_types=[
        pltpu.VMEM((_BPW,), jnp.int32),
        pltpu.VMEM((_WAVE, EMBED_DIM, 128), jnp.float32),
        pltpu.VMEM((EMBED_DIM, _BPW), jnp.float32),
        pltpu.SemaphoreType.DMA,
    ],
    compiler_params=pltpu.CompilerParams(needs_layout_passes=False),
)
def _gather_kernel(table_hbm, idx_hbm, out_hbm, idx_v, blk_v, out_v, sem):
    wid = lax.axis_index("s") * _NC + lax.axis_index("c")
    base = wid * _BPW
    pltpu.sync_copy(idx_hbm.at[wid], idx_v)
    lane = lax.iota(jnp.int32, LANES)

    def wave(w, carry):
        vgroups = [
            idx_v[pl.ds(w * _WAVE + g * LANES, LANES)]
            for g in range(_WAVE // LANES)
        ]
        copies = []
        for t in range(_WAVE):
            v = vgroups[t // LANES][t % LANES]
            off = pl.multiple_of(
                lax.shift_left(lax.shift_right_logical(v, 7), 7), 128
            )
            copies.append(
                pltpu.make_async_copy(
                    table_hbm.at[:, pl.ds(off, 128)], blk_v.at[t], sem
                )
            )
            copies[-1].start()
        for c in copies:
            c.wait()
        for t in range(_WAVE):
            i = w * _WAVE + t
            v = vgroups[t // LANES][t % LANES]
            col = jnp.full((LANES,), jnp.bitwise_and(v, 127), jnp.int32)
            vec = plsc.load_gather(blk_v.at[t], [lane, col])
            plsc.store_scatter(
                out_v, [lane, jnp.full((LANES,), i, jnp.int32)], vec
            )
        return carry

    lax.fori_loop(0, _NWAVE, wave, 0)
    pltpu.sync_copy(out_v, out_hbm.at[:, pl.ds(base, _BPW)])


def kernel(g, h, r, norm, node_attri):
    table_t = node_attri.T
    idx = h.reshape(_NW, _BPW)
    out_t = _gather_kernel(table_t, idx)
    return out_t.T


# speedup vs baseline: 5.5354x; 1.0008x over previous
import functools

import jax
import jax.numpy as jnp
from jax import lax
from jax.experimental import pallas as pl
from jax.experimental.pallas import tpu as pltpu
from jax.experimental.pallas import tpu_sc as plsc

EMBED_DIM = 16
BATCH = 16384
LANES = 16

_info = plsc.get_sparse_core_info()
_NC, _NS = _info.num_cores, _info.num_subcores
_NW = _NC * _NS
_BPW = BATCH // _NW
_WAVE = 32
_NWAVE = _BPW // _WAVE

_mesh = plsc.VectorSubcoreMesh(core_axis_name="c", subcore_axis_name="s")


@functools.partial(
    pl.kernel,
    mesh=_mesh,
    out_type=jax.ShapeDtypeStruct((EMBED_DIM, BATCH), jnp.float32),
    scratch_types=[
        pltpu.VMEM((_BPW,), jnp.int32),
        pltpu.VMEM((_WAVE, EMBED_DIM, 128), jnp.float32),
        pltpu.VMEM((EMBED_DIM, _BPW), jnp.float32),
        pltpu.SemaphoreType.DMA,
    ],
    compiler_params=pltpu.CompilerParams(needs_layout_passes=False),
)
def _gather_kernel(table_hbm, idx_hbm, out_hbm, idx_v, blk_v, out_v, sem):
    wid = lax.axis_index("s") * _NC + lax.axis_index("c")
    base = wid * _BPW
    pltpu.sync_copy(idx_hbm.at[pl.ds(base, _BPW)], idx_v)
    lane = lax.iota(jnp.int32, LANES)

    def wave(w, carry):
        vgroups = [
            idx_v[pl.ds(w * _WAVE + g * LANES, LANES)]
            for g in range(_WAVE // LANES)
        ]
        copies = []
        for t in range(_WAVE):
            v = vgroups[t // LANES][t % LANES]
            off = pl.multiple_of(
                lax.shift_left(lax.shift_right_logical(v, 7), 7), 128
            )
            copies.append(
                pltpu.make_async_copy(
                    table_hbm.at[:, pl.ds(off, 128)], blk_v.at[t], sem
                )
            )
            copies[-1].start()
        for c in copies:
            c.wait()
        for t in range(_WAVE):
            i = w * _WAVE + t
            v = vgroups[t // LANES][t % LANES]
            col = jnp.full((LANES,), jnp.bitwise_and(v, 127), jnp.int32)
            vec = plsc.load_gather(blk_v.at[t], [lane, col])
            plsc.store_scatter(
                out_v, [lane, jnp.full((LANES,), i, jnp.int32)], vec
            )
        return carry

    lax.fori_loop(0, _NWAVE, wave, 0)
    pltpu.sync_copy(out_v, out_hbm.at[:, pl.ds(base, _BPW)])


def kernel(g, h, r, norm, node_attri):
    table_t = node_attri.T
    idx = h.reshape(BATCH)
    out_t = _gather_kernel(table_t, idx)
    return out_t.T


# speedup vs baseline: 5.5372x; 1.0003x over previous
import functools

import jax
import jax.numpy as jnp
from jax import lax
from jax.experimental import pallas as pl
from jax.experimental.pallas import tpu as pltpu
from jax.experimental.pallas import tpu_sc as plsc

EMBED_DIM = 16
BATCH = 16384
LANES = 16

_info = plsc.get_sparse_core_info()
_NC, _NS = _info.num_cores, _info.num_subcores
_NW = _NC * _NS
_BPW = BATCH // _NW
_WAVE = 32
_NWAVE = _BPW // _WAVE

_mesh = plsc.VectorSubcoreMesh(core_axis_name="c", subcore_axis_name="s")


@functools.partial(
    pl.kernel,
    mesh=_mesh,
    out_type=jax.ShapeDtypeStruct((EMBED_DIM, BATCH), jnp.float32),
    scratch_types=[
        pltpu.VMEM((_BPW,), jnp.int32),
        pltpu.VMEM((_WAVE, EMBED_DIM, 128), jnp.float32),
        pltpu.VMEM((EMBED_DIM, _BPW), jnp.float32),
        pltpu.SemaphoreType.DMA,
    ],
    compiler_params=pltpu.CompilerParams(
        needs_layout_passes=False, skip_device_barrier=True
    ),
)
def _gather_kernel(table_hbm, idx_hbm, out_hbm, idx_v, blk_v, out_v, sem):
    wid = lax.axis_index("s") * _NC + lax.axis_index("c")
    base = wid * _BPW
    pltpu.sync_copy(idx_hbm.at[pl.ds(base, _BPW)], idx_v)
    lane = lax.iota(jnp.int32, LANES)

    def wave(w, carry):
        vgroups = [
            idx_v[pl.ds(w * _WAVE + g * LANES, LANES)]
            for g in range(_WAVE // LANES)
        ]
        copies = []
        for t in range(_WAVE):
            v = vgroups[t // LANES][t % LANES]
            off = pl.multiple_of(
                lax.shift_left(lax.shift_right_logical(v, 7), 7), 128
            )
            copies.append(
                pltpu.make_async_copy(
                    table_hbm.at[:, pl.ds(off, 128)], blk_v.at[t], sem
                )
            )
            copies[-1].start()
        for c in copies:
            c.wait()
        for t in range(_WAVE):
            i = w * _WAVE + t
            v = vgroups[t // LANES][t % LANES]
            col = jnp.full((LANES,), jnp.bitwise_and(v, 127), jnp.int32)
            vec = plsc.load_gather(blk_v.at[t], [lane, col])
            plsc.store_scatter(
                out_v, [lane, jnp.full((LANES,), i, jnp.int32)], vec
            )
        return carry

    lax.fori_loop(0, _NWAVE, wave, 0)
    pltpu.sync_copy(out_v, out_hbm.at[:, pl.ds(base, _BPW)])


def kernel(g, h, r, norm, node_attri):
    table_t = node_attri.T
    idx = h.reshape(BATCH)
    out_t = _gather_kernel(table_t, idx)
    return out_t.T


# speedup vs baseline: 6.0033x; 1.0842x over previous
import functools

import jax
import jax.numpy as jnp
from jax import lax
from jax.experimental import pallas as pl
from jax.experimental.pallas import tpu as pltpu
from jax.experimental.pallas import tpu_sc as plsc

EMBED_DIM = 16
BATCH = 16384
LANES = 16

_info = plsc.get_sparse_core_info()
_NC, _NS = _info.num_cores, _info.num_subcores
_NW = _NC * _NS
_BPW = BATCH // _NW
_WAVE = 16
_NWAVE = _BPW // _WAVE
_NITER = _NWAVE // 2

_mesh = plsc.VectorSubcoreMesh(core_axis_name="c", subcore_axis_name="s")


@functools.partial(
    pl.kernel,
    mesh=_mesh,
    out_type=jax.ShapeDtypeStruct((EMBED_DIM, BATCH), jnp.float32),
    scratch_types=[
        pltpu.VMEM((_BPW,), jnp.int32),
        pltpu.VMEM((2, _WAVE, EMBED_DIM, 128), jnp.float32),
        pltpu.VMEM((EMBED_DIM, _BPW), jnp.float32),
        pltpu.SemaphoreType.DMA,
        pltpu.SemaphoreType.DMA,
    ],
    compiler_params=pltpu.CompilerParams(needs_layout_passes=False),
)
def _gather_kernel(table_hbm, idx_hbm, out_hbm, idx_v, blk_v, out_v, sem_a, sem_b):
    wid = lax.axis_index("s") * _NC + lax.axis_index("c")
    base = wid * _BPW
    pltpu.sync_copy(idx_hbm.at[pl.ds(base, _BPW)], idx_v)
    lane = lax.iota(jnp.int32, LANES)

    def wave_idx(w):
        return idx_v[pl.ds(w * _WAVE, _WAVE)]

    def fire(w, buf, sem):
        vs = wave_idx(w)
        for t in range(_WAVE):
            off = pl.multiple_of(
                lax.shift_left(lax.shift_right_logical(vs[t], 7), 7), 128
            )
            pltpu.make_async_copy(
                table_hbm.at[:, pl.ds(off, 128)], blk_v.at[buf, t], sem
            ).start()

    def drain_extract(w, buf, sem):
        for t in range(_WAVE):
            pltpu.make_async_copy(
                table_hbm.at[:, pl.ds(0, 128)], blk_v.at[buf, t], sem
            ).wait()
        vs = wave_idx(w)
        for t in range(_WAVE):
            i = w * _WAVE + t
            col = jnp.full((LANES,), jnp.bitwise_and(vs[t], 127), jnp.int32)
            vec = plsc.load_gather(blk_v.at[buf, t], [lane, col])
            plsc.store_scatter(
                out_v, [lane, jnp.full((LANES,), i, jnp.int32)], vec
            )

    fire(0, 0, sem_a)

    def body(k, carry):
        fire(2 * k + 1, 1, sem_b)
        drain_extract(2 * k, 0, sem_a)

        @pl.when(k < _NITER - 1)
        def _():
            fire(2 * k + 2, 0, sem_a)

        drain_extract(2 * k + 1, 1, sem_b)
        return carry

    lax.fori_loop(0, _NITER, body, 0)
    pltpu.sync_copy(out_v, out_hbm.at[:, pl.ds(base, _BPW)])


def kernel(g, h, r, norm, node_attri):
    table_t = node_attri.T
    idx = h.reshape(BATCH)
    out_t = _gather_kernel(table_t, idx)
    return out_t.T


# speedup vs baseline: 6.3527x; 1.0582x over previous
import functools

import jax
import jax.numpy as jnp
from jax import lax
from jax.experimental import pallas as pl
from jax.experimental.pallas import tpu as pltpu
from jax.experimental.pallas import tpu_sc as plsc

EMBED_DIM = 16
BATCH = 16384
LANES = 16

_info = plsc.get_sparse_core_info()
_NC, _NS = _info.num_cores, _info.num_subcores
_NW = _NC * _NS
_BPW = BATCH // _NW
_WAVE = 16
_NWAVE = _BPW // _WAVE
_NITER = _NWAVE // 2

_mesh = plsc.VectorSubcoreMesh(core_axis_name="c", subcore_axis_name="s")


@functools.partial(
    pl.kernel,
    mesh=_mesh,
    out_type=jax.ShapeDtypeStruct((EMBED_DIM, BATCH), jnp.float32),
    scratch_types=[
        pltpu.VMEM((_BPW,), jnp.int32),
        pltpu.VMEM((3, _WAVE, EMBED_DIM, 128), jnp.float32),
        pltpu.VMEM((EMBED_DIM, _BPW), jnp.float32),
        pltpu.SemaphoreType.DMA,
        pltpu.SemaphoreType.DMA,
        pltpu.SemaphoreType.DMA,
    ],
    compiler_params=pltpu.CompilerParams(needs_layout_passes=False),
)
def _gather_kernel(table_hbm, idx_hbm, out_hbm, idx_v, blk_v, out_v, s0, s1, s2):
    wid = lax.axis_index("s") * _NC + lax.axis_index("c")
    base = wid * _BPW
    pltpu.sync_copy(idx_hbm.at[pl.ds(base, _BPW)], idx_v)
    lane = lax.iota(jnp.int32, LANES)

    def wave_idx(w):
        return idx_v[pl.ds(w * _WAVE, _WAVE)]

    def fire(w, buf, sem):
        offs = lax.shift_left(lax.shift_right_logical(wave_idx(w), 7), 7)
        for t in range(_WAVE):
            off = pl.multiple_of(offs[t], 128)
            pltpu.make_async_copy(
                table_hbm.at[:, pl.ds(off, 128)], blk_v.at[buf, t], sem
            ).start()

    def drain_extract(w, buf, sem):
        for t in range(_WAVE):
            pltpu.make_async_copy(
                table_hbm.at[:, pl.ds(0, 128)], blk_v.at[buf, t], sem
            ).wait()
        cols = jnp.bitwise_and(wave_idx(w), 127)
        for t in range(_WAVE):
            i = w * _WAVE + t
            col = jnp.full((LANES,), cols[t], jnp.int32)
            vec = plsc.load_gather(blk_v.at[buf, t], [lane, col])
            plsc.store_scatter(
                out_v, [lane, jnp.full((LANES,), i, jnp.int32)], vec
            )

    fire(0, 0, s0)
    fire(1, 1, s1)

    def body(j, carry):
        w = 3 * j
        fire(w + 2, 2, s2)
        drain_extract(w, 0, s0)
        fire(w + 3, 0, s0)
        drain_extract(w + 1, 1, s1)
        fire(w + 4, 1, s1)
        drain_extract(w + 2, 2, s2)
        return carry

    lax.fori_loop(0, (_NWAVE - 2) // 3, body, 0)
    drain_extract(_NWAVE - 2, 0, s0)
    drain_extract(_NWAVE - 1, 1, s1)
    pltpu.sync_copy(out_v, out_hbm.at[:, pl.ds(base, _BPW)])


def kernel(g, h, r, norm, node_attri):
    table_t = node_attri.T
    idx = h.reshape(BATCH)
    out_t = _gather_kernel(table_t, idx)
    return out_t.T
